# baseline (speedup 1.0000x reference)
import jax
import jax.numpy as jnp
from jax.experimental import pallas as pl
from jax.experimental.pallas import tpu as pltpu


def _round_up(x, m):
    return ((x + m - 1) // m) * m


def _mm_bias_kernel(x_ref, w_ref, b_ref, o_ref):
    xb = x_ref[...].astype(jnp.bfloat16)
    acc = jnp.dot(xb, w_ref[...], preferred_element_type=jnp.float32)
    o_ref[...] = (acc + b_ref[...].astype(jnp.float32)).astype(o_ref.dtype)


def kernel(x_bcl, weight, bias):
    B, C, L_in = x_bcl.shape
    L_out = weight.shape[-1]
    M = B * C

    x2 = x_bcl.reshape(M, L_in)
    w = weight.reshape(L_in, L_out).astype(jnp.bfloat16)
    b = bias.reshape(1, L_out)

    bm = 1024
    M_pad = _round_up(M, bm)
    if M_pad != M:
        x2 = jnp.pad(x2, ((0, M_pad - M), (0, 0)))

    out = pl.pallas_call(
        _mm_bias_kernel,
        out_shape=jax.ShapeDtypeStruct((M_pad, L_out), x_bcl.dtype),
        grid=(M_pad // bm,),
        in_specs=[
            pl.BlockSpec((bm, L_in), lambda i: (i, 0)),
            pl.BlockSpec((L_in, L_out), lambda i: (0, 0)),
            pl.BlockSpec((1, L_out), lambda i: (0, 0)),
        ],
        out_specs=pl.BlockSpec((bm, L_out), lambda i: (i, 0)),
        compiler_params=pltpu.CompilerParams(
            dimension_semantics=("parallel",),
            vmem_limit_bytes=64 * 1024 * 1024,
        ),
    )(x2, w, b)

    return out[:M].reshape(B, C, L_out)


# speedup vs baseline: 1.2154x; 1.2154x over previous
import jax
import jax.numpy as jnp
from jax.experimental import pallas as pl
from jax.experimental.pallas import tpu as pltpu


def _round_up(x, m):
    return ((x + m - 1) // m) * m


def _mm_bias_kernel(x_ref, w_ref, b_ref, o_ref):
    xb = x_ref[...].astype(jnp.bfloat16)
    acc = jnp.dot(xb, w_ref[...], preferred_element_type=jnp.float32)
    o_ref[...] = (acc + b_ref[...].astype(jnp.float32)).astype(o_ref.dtype)


def kernel(x_bcl, weight, bias):
    B, C, L_in = x_bcl.shape
    L_out = weight.shape[-1]
    M = B * C

    x2 = x_bcl.reshape(M, L_in)
    w = weight.reshape(L_in, L_out).astype(jnp.bfloat16)
    b = bias.reshape(1, L_out)

    bm = 4096
    M_pad = _round_up(M, bm)
    if M_pad != M:
        x2 = jnp.pad(x2, ((0, M_pad - M), (0, 0)))

    out = pl.pallas_call(
        _mm_bias_kernel,
        out_shape=jax.ShapeDtypeStruct((M_pad, L_out), x_bcl.dtype),
        grid=(M_pad // bm,),
        in_specs=[
            pl.BlockSpec((bm, L_in), lambda i: (i, 0)),
            pl.BlockSpec((L_in, L_out), lambda i: (0, 0)),
            pl.BlockSpec((1, L_out), lambda i: (0, 0)),
        ],
        out_specs=pl.BlockSpec((bm, L_out), lambda i: (i, 0)),
        compiler_params=pltpu.CompilerParams(
            dimension_semantics=("parallel",),
            vmem_limit_bytes=64 * 1024 * 1024,
        ),
    )(x2, w, b)

    return out[:M].reshape(B, C, L_out)


# speedup vs baseline: 1.2727x; 1.0472x over previous
import jax
import jax.numpy as jnp
from jax.experimental import pallas as pl
from jax.experimental.pallas import tpu as pltpu


def _round_up(x, m):
    return ((x + m - 1) // m) * m


def _mm_bias_kernel(x_ref, w_ref, b_ref, o_ref):
    xb = x_ref[...].astype(jnp.bfloat16)
    wb = w_ref[...].astype(jnp.bfloat16)
    acc = jnp.dot(xb, wb, preferred_element_type=jnp.float32)
    o_ref[...] = (acc + b_ref[...].astype(jnp.float32)).astype(o_ref.dtype)


def kernel(x_bcl, weight, bias):
    B, C, L_in = x_bcl.shape
    L_out = weight.shape[-1]
    M = B * C

    x2 = x_bcl.reshape(M, L_in)
    w = weight.reshape(L_in, L_out)
    b = bias.reshape(1, L_out)

    bm = 4096
    M_pad = _round_up(M, bm)
    if M_pad != M:
        x2 = jnp.pad(x2, ((0, M_pad - M), (0, 0)))

    out = pl.pallas_call(
        _mm_bias_kernel,
        out_shape=jax.ShapeDtypeStruct((M_pad, L_out), x_bcl.dtype),
        grid=(M_pad // bm,),
        in_specs=[
            pl.BlockSpec((bm, L_in), lambda i: (i, 0)),
            pl.BlockSpec((L_in, L_out), lambda i: (0, 0)),
            pl.BlockSpec((1, L_out), lambda i: (0, 0)),
        ],
        out_specs=pl.BlockSpec((bm, L_out), lambda i: (i, 0)),
        compiler_params=pltpu.CompilerParams(
            dimension_semantics=("parallel",),
            vmem_limit_bytes=64 * 1024 * 1024,
        ),
    )(x2, w, b)

    return out[:M].reshape(B, C, L_out)
